# baseline (speedup 1.0000x reference)
import functools

import jax
import jax.numpy as jnp
from jax import lax
from jax.experimental import pallas as pl
from jax.experimental.pallas import tpu as pltpu
from jax.experimental.pallas import tpu_sc as plsc

BATCH = 16384
EMBED_DIM = 64
NUM_CORES = 2
NUM_SUBCORES = 16
NUM_WORKERS = NUM_CORES * NUM_SUBCORES
CHUNK = 128
ROWS_PER_WORKER = BATCH // NUM_WORKERS
CHUNKS_PER_WORKER = ROWS_PER_WORKER // CHUNK

_mesh = plsc.VectorSubcoreMesh(core_axis_name="c", subcore_axis_name="s")


@functools.partial(
    pl.kernel,
    out_type=jax.ShapeDtypeStruct((BATCH, EMBED_DIM), jnp.float32),
    mesh=_mesh,
    compiler_params=pltpu.CompilerParams(use_tc_tiling_on_sc=False),
    scratch_types=[
        pltpu.VMEM((CHUNKS_PER_WORKER, CHUNK), jnp.int32),
        pltpu.VMEM((CHUNK, EMBED_DIM), jnp.float32),
    ],
)
def _encoder_kernel(idx_hbm, item_hbm, ent_hbm, out_hbm, idx_v, rows_v):
    wid = lax.axis_index("s") * NUM_CORES + lax.axis_index("c")
    pltpu.sync_copy(idx_hbm.at[wid], idx_v)
    base = wid * ROWS_PER_WORKER
    for j in range(CHUNKS_PER_WORKER):
        pltpu.sync_copy(item_hbm.at[idx_v.at[j]], rows_v)
        pltpu.sync_copy(ent_hbm.at[idx_v.at[j]], rows_v, add=True)
        pltpu.sync_copy(rows_v, out_hbm.at[pl.ds(base + j * CHUNK, CHUNK)])


def kernel(batch_data, item_table, ent_table):
    idx = batch_data.astype(jnp.int32).reshape(NUM_WORKERS, CHUNKS_PER_WORKER, CHUNK)
    return _encoder_kernel(idx, item_table, ent_table)


# speedup vs baseline: 1.0025x; 1.0025x over previous
import functools

import jax
import jax.numpy as jnp
from jax import lax
from jax.experimental import pallas as pl
from jax.experimental.pallas import tpu as pltpu
from jax.experimental.pallas import tpu_sc as plsc

BATCH = 16384
EMBED_DIM = 64
NUM_CORES = 2
NUM_SUBCORES = 16
NUM_WORKERS = NUM_CORES * NUM_SUBCORES
CHUNK = 128
ROWS_PER_WORKER = BATCH // NUM_WORKERS
CHUNKS_PER_WORKER = ROWS_PER_WORKER // CHUNK

_mesh = plsc.VectorSubcoreMesh(core_axis_name="c", subcore_axis_name="s")


@functools.partial(
    pl.kernel,
    out_type=jax.ShapeDtypeStruct((BATCH, EMBED_DIM), jnp.float32),
    mesh=_mesh,
    compiler_params=pltpu.CompilerParams(use_tc_tiling_on_sc=False),
    scratch_types=[
        pltpu.VMEM((CHUNKS_PER_WORKER, CHUNK), jnp.int32),
        pltpu.VMEM((CHUNKS_PER_WORKER, CHUNK, EMBED_DIM), jnp.float32),
        pltpu.SemaphoreType.DMA((CHUNKS_PER_WORKER,)),
        pltpu.SemaphoreType.DMA((CHUNKS_PER_WORKER,)),
        pltpu.SemaphoreType.DMA((CHUNKS_PER_WORKER,)),
    ],
)
def _encoder_kernel(idx_hbm, item_hbm, ent_hbm, out_hbm, idx_v, rows_v,
                    sem_g, sem_a, sem_w):
    wid = lax.axis_index("s") * NUM_CORES + lax.axis_index("c")
    pltpu.sync_copy(idx_hbm.at[wid], idx_v)
    base = wid * ROWS_PER_WORKER

    gathers = [
        pltpu.async_copy(item_hbm.at[idx_v.at[j]], rows_v.at[j], sem_g.at[j])
        for j in range(CHUNKS_PER_WORKER)
    ]
    adds = []
    for j in range(CHUNKS_PER_WORKER):
        gathers[j].wait()
        adds.append(
            pltpu.async_copy(ent_hbm.at[idx_v.at[j]], rows_v.at[j],
                             sem_a.at[j], add=True))
    writes = []
    for j in range(CHUNKS_PER_WORKER):
        adds[j].wait()
        writes.append(
            pltpu.async_copy(rows_v.at[j],
                             out_hbm.at[pl.ds(base + j * CHUNK, CHUNK)],
                             sem_w.at[j]))
    for w in writes:
        w.wait()


def kernel(batch_data, item_table, ent_table):
    idx = batch_data.astype(jnp.int32).reshape(NUM_WORKERS, CHUNKS_PER_WORKER, CHUNK)
    return _encoder_kernel(idx, item_table, ent_table)


# speedup vs baseline: 1.5737x; 1.5697x over previous
import functools

import jax
import jax.numpy as jnp
from jax import lax
from jax.experimental import pallas as pl
from jax.experimental.pallas import tpu as pltpu
from jax.experimental.pallas import tpu_sc as plsc

BATCH = 16384
EMBED_DIM = 64
NUM_CORES = 2
NUM_SUBCORES = 16
NUM_WORKERS = NUM_CORES * NUM_SUBCORES
ROWS_PER_WORKER = BATCH // NUM_WORKERS
CHUNK = 128
NCHUNKS = ROWS_PER_WORKER // CHUNK
LANES = 16

_mesh = plsc.VectorSubcoreMesh(core_axis_name="c", subcore_axis_name="s")


@functools.partial(
    pl.kernel,
    out_type=jax.ShapeDtypeStruct((BATCH, EMBED_DIM), jnp.float32),
    mesh=_mesh,
    scratch_types=[
        pltpu.VMEM((ROWS_PER_WORKER,), jnp.int32),
        pltpu.VMEM((CHUNK, EMBED_DIM), jnp.float32),
        pltpu.VMEM((CHUNK, EMBED_DIM), jnp.float32),
        pltpu.SemaphoreType.DMA,
        pltpu.SemaphoreType.DMA,
        pltpu.SemaphoreType.DMA,
    ],
)
def _encoder_kernel(idx_hbm, item_hbm, ent_hbm, out_hbm, idx_v,
                    buf_a, buf_b, sem_i, sem_a, sem_b):
    wid = lax.axis_index("s") * NUM_CORES + lax.axis_index("c")
    base = wid * ROWS_PER_WORKER
    pltpu.async_copy(idx_hbm.at[wid], idx_v, sem_i).wait()

    @pl.loop(0, NCHUNKS)
    def _chunk(chunk):
        off = chunk * CHUNK
        for v in range(CHUNK // LANES):
            vec = idx_v[pl.ds(off + v * LANES, LANES)]
            for l in range(LANES):
                r = vec[l]
                i = v * LANES + l
                pltpu.async_copy(item_hbm.at[pl.ds(r, 1)],
                                 buf_a.at[pl.ds(i, 1)], sem_a)
                pltpu.async_copy(ent_hbm.at[pl.ds(r, 1)],
                                 buf_b.at[pl.ds(i, 1)], sem_b)

        @pl.loop(0, CHUNK)
        def _drain(i):
            pltpu.make_async_copy(item_hbm.at[pl.ds(0, 1)],
                                  buf_a.at[pl.ds(i, 1)], sem_a).wait()
            pltpu.make_async_copy(ent_hbm.at[pl.ds(0, 1)],
                                  buf_b.at[pl.ds(i, 1)], sem_b).wait()

        @pl.loop(0, CHUNK)
        def _add(i):
            for c in range(EMBED_DIM // LANES):
                sl = pl.ds(c * LANES, LANES)
                buf_a[i, sl] = buf_a[i, sl] + buf_b[i, sl]

        pltpu.sync_copy(buf_a, out_hbm.at[pl.ds(base + off, CHUNK)])


def kernel(batch_data, item_table, ent_table):
    idx = batch_data.astype(jnp.int32).reshape(NUM_WORKERS, ROWS_PER_WORKER)
    return _encoder_kernel(idx, item_table, ent_table)


# speedup vs baseline: 1.5741x; 1.0003x over previous
import functools

import jax
import jax.numpy as jnp
from jax import lax
from jax.experimental import pallas as pl
from jax.experimental.pallas import tpu as pltpu
from jax.experimental.pallas import tpu_sc as plsc

BATCH = 16384
EMBED_DIM = 64
NUM_CORES = 2
NUM_SUBCORES = 16
NUM_WORKERS = NUM_CORES * NUM_SUBCORES
ROWS_PER_WORKER = BATCH // NUM_WORKERS
CHUNK = 128
NCHUNKS = ROWS_PER_WORKER // CHUNK
LANES = 16

_mesh = plsc.VectorSubcoreMesh(core_axis_name="c", subcore_axis_name="s")


@functools.partial(
    pl.kernel,
    out_type=jax.ShapeDtypeStruct((BATCH, EMBED_DIM), jnp.float32),
    mesh=_mesh,
    compiler_params=pltpu.CompilerParams(use_tc_tiling_on_sc=True),
    scratch_types=[
        pltpu.VMEM((ROWS_PER_WORKER,), jnp.int32),
        pltpu.VMEM((CHUNK, EMBED_DIM), jnp.float32),
        pltpu.VMEM((CHUNK, EMBED_DIM), jnp.float32),
        pltpu.SemaphoreType.DMA,
        pltpu.SemaphoreType.DMA,
        pltpu.SemaphoreType.DMA,
    ],
)
def _encoder_kernel(idx_hbm, item_hbm, ent_hbm, out_hbm, idx_v,
                    buf_a, buf_b, sem_i, sem_a, sem_b):
    wid = lax.axis_index("s") * NUM_CORES + lax.axis_index("c")
    base = wid * ROWS_PER_WORKER
    pltpu.async_copy(idx_hbm.at[wid], idx_v, sem_i).wait()

    @pl.loop(0, NCHUNKS)
    def _chunk(chunk):
        off = chunk * CHUNK
        for v in range(CHUNK // LANES):
            vec = idx_v[pl.ds(off + v * LANES, LANES)]
            for l in range(LANES):
                r = vec[l]
                i = v * LANES + l
                pltpu.async_copy(item_hbm.at[pl.ds(r, 1)],
                                 buf_a.at[pl.ds(i, 1)], sem_a)
                pltpu.async_copy(ent_hbm.at[pl.ds(r, 1)],
                                 buf_b.at[pl.ds(i, 1)], sem_b)

        @pl.loop(0, CHUNK)
        def _drain(i):
            pltpu.make_async_copy(item_hbm.at[pl.ds(0, 1)],
                                  buf_a.at[pl.ds(i, 1)], sem_a).wait()
            pltpu.make_async_copy(ent_hbm.at[pl.ds(0, 1)],
                                  buf_b.at[pl.ds(i, 1)], sem_b).wait()

        @pl.loop(0, CHUNK)
        def _add(i):
            for c in range(EMBED_DIM // LANES):
                sl = pl.ds(c * LANES, LANES)
                buf_a[i, sl] = buf_a[i, sl] + buf_b[i, sl]

        pltpu.sync_copy(buf_a, out_hbm.at[pl.ds(base + off, CHUNK)])


def kernel(batch_data, item_table, ent_table):
    idx = batch_data.astype(jnp.int32).reshape(NUM_WORKERS, ROWS_PER_WORKER)
    return _encoder_kernel(idx, item_table, ent_table)
